# baseline (speedup 1.0000x reference)
import functools

import jax
import jax.numpy as jnp
import numpy as np
from jax import lax
from jax.experimental import pallas as pl
from jax.experimental.pallas import tpu as pltpu
from jax.experimental.pallas import tpu_sc as plsc

NC = 2
NS = 16
NW = NC * NS
LANES = 16

B = 4096
L = 200
D = 64
EPW = B // NW
NEX = 2
NG = EPW // NEX

_PERM = np.concatenate([np.arange(0, 32, 2), np.arange(1, 32, 2),
                        np.arange(32, 64, 2), np.arange(33, 64, 2)])
_INV = np.argsort(_PERM)
_HI_MASK = np.int32(-65536)


def _sc_body(gids_hbm, cnts_hbm, table_hbm, out_hbm,
             gid_v, cnt_v, rows_v, out_v, sem0, sem1):
    wid = lax.axis_index("s") * NC + lax.axis_index("c")
    base = wid * EPW

    pltpu.sync_copy(gids_hbm.at[pl.ds(base * L, EPW * L)], gid_v)
    pltpu.sync_copy(cnts_hbm.at[pl.ds(base, EPW)], cnt_v)

    sems = (sem0, sem1)

    def gather_desc(g, b):
        return pltpu.make_async_copy(
            table_hbm.at[gid_v.at[pl.ds(g * (NEX * L), NEX * L)]],
            rows_v.at[b], sems[b])

    gather_desc(0, 0).start()
    gather_desc(1, 1).start()

    def outer(i, carry):
        for b in range(2):
            g = 2 * i + b
            gather_desc(g, b).wait()

            for n in range(NEX):
                e = g * NEX + n

                def inner(k, acc, b=b, n=n, e=e):
                    a0, a1, a2, a3, sv = acc
                    w16 = cnt_v[e, pl.ds(k * LANES, LANES)]
                    sv = sv + w16
                    for j in range(LANES):
                        w = w16[j]
                        r = n * L + k * LANES + j
                        x0 = rows_v[b, r, pl.ds(0, LANES)]
                        x1 = rows_v[b, r, pl.ds(LANES, LANES)]
                        e0 = lax.bitcast_convert_type(x0 << 16, jnp.float32)
                        o0 = lax.bitcast_convert_type(x0 & _HI_MASK, jnp.float32)
                        e1 = lax.bitcast_convert_type(x1 << 16, jnp.float32)
                        o1 = lax.bitcast_convert_type(x1 & _HI_MASK, jnp.float32)
                        a0 = a0 + w * e0
                        a1 = a1 + w * o0
                        a2 = a2 + w * e1
                        a3 = a3 + w * o1
                    return (a0, a1, a2, a3, sv)

                z = jnp.zeros((LANES,), jnp.float32)
                a0, a1, a2, a3, sv = lax.fori_loop(
                    0, L // LANES, inner, (z, z, z, z, z))

                w16 = cnt_v[e, pl.ds(L - LANES, LANES)]
                tail_mask = lax.iota(jnp.int32, LANES) >= (LANES - (L % LANES))
                sv = sv + jnp.where(tail_mask, w16, 0.0)
                for j in range(LANES - (L % LANES), LANES):
                    w = w16[j]
                    r = n * L + (L - LANES) + j
                    x0 = rows_v[b, r, pl.ds(0, LANES)]
                    x1 = rows_v[b, r, pl.ds(LANES, LANES)]
                    e0 = lax.bitcast_convert_type(x0 << 16, jnp.float32)
                    o0 = lax.bitcast_convert_type(x0 & _HI_MASK, jnp.float32)
                    e1 = lax.bitcast_convert_type(x1 << 16, jnp.float32)
                    o1 = lax.bitcast_convert_type(x1 & _HI_MASK, jnp.float32)
                    a0 = a0 + w * e0
                    a1 = a1 + w * o0
                    a2 = a2 + w * e1
                    a3 = a3 + w * o1

                lane = lax.iota(jnp.int32, LANES)
                dnums = lax.GatherDimensionNumbers(
                    offset_dims=(), collapsed_slice_dims=(0,),
                    start_index_map=(0,))
                for s in (1, 2, 4, 8):
                    perm = (lane ^ s).reshape(LANES, 1)
                    sv = sv + lax.gather(
                        sv, perm, dnums, (1,),
                        mode=lax.GatherScatterMode.PROMISE_IN_BOUNDS)
                inv = 1.0 / sv
                out_v[e, pl.ds(0, LANES)] = a0 * inv
                out_v[e, pl.ds(LANES, LANES)] = a1 * inv
                out_v[e, pl.ds(2 * LANES, LANES)] = a2 * inv
                out_v[e, pl.ds(3 * LANES, LANES)] = a3 * inv

            @pl.when(g + 2 < NG)
            def _(g=g, b=b):
                gather_desc(g + 2, b).start()
        return carry

    lax.fori_loop(0, NG // 2, outer, 0)
    pltpu.sync_copy(out_v, out_hbm.at[pl.ds(base, EPW)])


_sc_embed = functools.partial(
    pl.kernel,
    mesh=plsc.VectorSubcoreMesh(core_axis_name="c", subcore_axis_name="s"),
    out_type=jax.ShapeDtypeStruct((B, D), jnp.float32),
    compiler_params=pltpu.CompilerParams(use_tc_tiling_on_sc=False),
    scratch_types=[
        pltpu.VMEM((EPW * L,), jnp.int32),
        pltpu.VMEM((EPW, L), jnp.float32),
        pltpu.VMEM((2, NEX * L, D // 2), jnp.int32),
        pltpu.VMEM((EPW, D), jnp.float32),
        pltpu.SemaphoreType.DMA,
        pltpu.SemaphoreType.DMA,
    ],
)(_sc_body)


def kernel(gids, cnts, table):
    assert gids.shape == (B, L) and cnts.shape == (B, L)
    assert table.shape[1] == D
    gids_f = gids.astype(jnp.int32).reshape(B * L)
    cnts = cnts.astype(jnp.float32)
    table16 = table.astype(jnp.bfloat16)
    table_pk = jax.lax.bitcast_convert_type(
        table16.reshape(table.shape[0], D // 2, 2), jnp.int32)
    out = _sc_embed(gids_f, cnts, table_pk)
    return out[:, _INV]


# speedup vs baseline: 2.3061x; 2.3061x over previous
import functools

import jax
import jax.numpy as jnp
from jax import lax
from jax.experimental import pallas as pl
from jax.experimental.pallas import tpu as pltpu
from jax.experimental.pallas import tpu_sc as plsc

NC = 2
NS = 16
NW = NC * NS
LANES = 16

B = 4096
L = 200
D = 64
EPW = B // NW
NEX = 4
NG = EPW // NEX


def _sc_body(gids_hbm, cnts_hbm, table_hbm, out_hbm,
             gid_v, cnt_v, rows_v, out_v,
             gsem0, gsem1, isem0, isem1, csem0, csem1):
    wid = lax.axis_index("s") * NC + lax.axis_index("c")
    base = wid * EPW

    gsems = (gsem0, gsem1)
    isems = (isem0, isem1)
    csems = (csem0, csem1)

    def gid_desc(g, b):
        return pltpu.make_async_copy(
            gids_hbm.at[pl.ds((base + g * NEX) * L, NEX * L)],
            gid_v.at[b], isems[b])

    def cnt_desc(g, b):
        return pltpu.make_async_copy(
            cnts_hbm.at[pl.ds(base + g * NEX, NEX)], cnt_v.at[b], csems[b])

    def gather_desc(g, b):
        return pltpu.make_async_copy(
            table_hbm.at[gid_v.at[b]], rows_v.at[b], gsems[b])

    gid_desc(0, 0).start()
    gid_desc(1, 1).start()
    cnt_desc(0, 0).start()
    cnt_desc(1, 1).start()
    gid_desc(0, 0).wait()
    gather_desc(0, 0).start()
    gid_desc(1, 1).wait()
    gather_desc(1, 1).start()

    def outer(i, carry):
        for b in range(2):
            g = 2 * i + b
            gather_desc(g, b).wait()

            @pl.when(g + 2 < NG)
            def _(g=g, b=b):
                gid_desc(g + 2, b).start()

            cnt_desc(g, b).wait()

            for n in range(NEX):
                e = g * NEX + n

                def inner(k, acc, b=b, n=n):
                    a0, a1, a2, a3, sv = acc
                    w16 = cnt_v[b, n, pl.ds(k * LANES, LANES)]
                    sv = sv + w16
                    for j in range(LANES):
                        w = w16[j]
                        r = n * L + k * LANES + j
                        a0 = a0 + w * rows_v[b, r, pl.ds(0, LANES)]
                        a1 = a1 + w * rows_v[b, r, pl.ds(LANES, LANES)]
                        a2 = a2 + w * rows_v[b, r, pl.ds(2 * LANES, LANES)]
                        a3 = a3 + w * rows_v[b, r, pl.ds(3 * LANES, LANES)]
                    return (a0, a1, a2, a3, sv)

                z = jnp.zeros((LANES,), jnp.float32)
                a0, a1, a2, a3, sv = lax.fori_loop(
                    0, L // LANES, inner, (z, z, z, z, z))

                w16 = cnt_v[b, n, pl.ds(L - LANES, LANES)]
                tail_mask = lax.iota(jnp.int32, LANES) >= (LANES - (L % LANES))
                sv = sv + jnp.where(tail_mask, w16, 0.0)
                for j in range(LANES - (L % LANES), LANES):
                    w = w16[j]
                    r = n * L + (L - LANES) + j
                    a0 = a0 + w * rows_v[b, r, pl.ds(0, LANES)]
                    a1 = a1 + w * rows_v[b, r, pl.ds(LANES, LANES)]
                    a2 = a2 + w * rows_v[b, r, pl.ds(2 * LANES, LANES)]
                    a3 = a3 + w * rows_v[b, r, pl.ds(3 * LANES, LANES)]

                lane = lax.iota(jnp.int32, LANES)
                dnums = lax.GatherDimensionNumbers(
                    offset_dims=(), collapsed_slice_dims=(0,),
                    start_index_map=(0,))
                for s in (1, 2, 4, 8):
                    perm = (lane ^ s).reshape(LANES, 1)
                    sv = sv + lax.gather(
                        sv, perm, dnums, (1,),
                        mode=lax.GatherScatterMode.PROMISE_IN_BOUNDS)
                inv = 1.0 / sv
                out_v[e, pl.ds(0, LANES)] = a0 * inv
                out_v[e, pl.ds(LANES, LANES)] = a1 * inv
                out_v[e, pl.ds(2 * LANES, LANES)] = a2 * inv
                out_v[e, pl.ds(3 * LANES, LANES)] = a3 * inv

            @pl.when(g + 2 < NG)
            def _(g=g, b=b):
                cnt_desc(g + 2, b).start()
                gid_desc(g + 2, b).wait()
                gather_desc(g + 2, b).start()
        return carry

    lax.fori_loop(0, NG // 2, outer, 0)
    pltpu.sync_copy(out_v, out_hbm.at[pl.ds(base, EPW)])


_sc_embed = functools.partial(
    pl.kernel,
    mesh=plsc.VectorSubcoreMesh(core_axis_name="c", subcore_axis_name="s"),
    out_type=jax.ShapeDtypeStruct((B, D), jnp.float32),
    compiler_params=pltpu.CompilerParams(use_tc_tiling_on_sc=False),
    scratch_types=[
        pltpu.VMEM((2, NEX * L), jnp.int32),
        pltpu.VMEM((2, NEX, L), jnp.float32),
        pltpu.VMEM((2, NEX * L, D), jnp.float32),
        pltpu.VMEM((EPW, D), jnp.float32),
        pltpu.SemaphoreType.DMA,
        pltpu.SemaphoreType.DMA,
        pltpu.SemaphoreType.DMA,
        pltpu.SemaphoreType.DMA,
        pltpu.SemaphoreType.DMA,
        pltpu.SemaphoreType.DMA,
    ],
)(_sc_body)


def kernel(gids, cnts, table):
    assert gids.shape == (B, L) and cnts.shape == (B, L)
    assert table.shape[1] == D
    gids_f = gids.astype(jnp.int32).reshape(B * L)
    cnts = cnts.astype(jnp.float32)
    table = table.astype(jnp.float32)
    return _sc_embed(gids_f, cnts, table)
